# baseline (speedup 1.0000x reference)
import functools

import jax
import jax.numpy as jnp
from jax import lax
from jax.experimental import pallas as pl
from jax.experimental.pallas import tpu as pltpu
from jax.experimental.pallas import tpu_sc as plsc

N_NODES = 100000
N_EDGES = 3200000

NW = 32
EPW = N_EDGES // NW
C = 10000
NCH = EPW // C

NPAD = 100352
NPT = NPAD // 16
TC_R = NPAD // 128
LANES = 128

_MESH = plsc.VectorSubcoreMesh(core_axis_name="c", subcore_axis_name="s")
_f32 = jnp.float32


@functools.partial(
    pl.kernel,
    out_type=jax.ShapeDtypeStruct((2 * NPAD,), _f32),
    mesh=_MESH,
    scratch_types=[
        pltpu.VMEM((C,), jnp.int32),
        pltpu.VMEM((C,), jnp.int32),
        pltpu.VMEM((C,), _f32),
        pltpu.VMEM_SHARED((NPAD,), _f32),
        pltpu.SemaphoreType.DMA,
        pltpu.SemaphoreType.DMA,
    ],
)
def _deg_kernel(dst_hbm, ones_hbm, zeros_hbm, out_hbm,
                idx0_v, idx1_v, ones_v, acc_sh, sc0, sc1):
    cid = lax.axis_index("c")
    sid = lax.axis_index("s")
    w = sid * 2 + cid
    off = sid * NPT
    pltpu.sync_copy(ones_hbm, ones_v)
    pltpu.sync_copy(zeros_hbm.at[pl.ds(off, NPT)], acc_sh.at[pl.ds(off, NPT)])
    plsc.subcore_barrier()

    e0 = w * EPW
    bufs = ((idx0_v, sc0), (idx1_v, sc1))

    def body(i, carry):
        for b, (idx_v, sc) in enumerate(bufs):
            @pl.when(i > 0)
            def _():
                pltpu.make_async_copy(ones_v, acc_sh.at[idx_v], sc).wait()

            pltpu.sync_copy(dst_hbm.at[pl.ds(e0 + (2 * i + b) * C, C)], idx_v)
            pltpu.async_copy(ones_v, acc_sh.at[idx_v], sc, add=True)
        return carry

    lax.fori_loop(0, NCH // 2, body, 0)
    for idx_v, sc in bufs:
        pltpu.make_async_copy(ones_v, acc_sh.at[idx_v], sc).wait()

    plsc.subcore_barrier()
    pltpu.sync_copy(acc_sh.at[pl.ds(off, NPT)],
                    out_hbm.at[pl.ds(cid * NPAD + off, NPT)])


@functools.partial(
    pl.kernel,
    out_type=jax.ShapeDtypeStruct((2 * NPAD,), _f32),
    mesh=_MESH,
    scratch_types=[
        pltpu.VMEM((C,), jnp.int32),
        pltpu.VMEM((C,), jnp.int32),
        pltpu.VMEM((C,), jnp.int32),
        pltpu.VMEM((C,), jnp.int32),
        pltpu.VMEM((C,), _f32),
        pltpu.VMEM((C,), _f32),
        pltpu.VMEM_SHARED((NPAD,), _f32),
        pltpu.VMEM_SHARED((NPAD,), _f32),
        pltpu.SemaphoreType.DMA,
        pltpu.SemaphoreType.DMA,
    ],
)
def _prop1_kernel(src_hbm, dst_hbm, w_hbm, zeros_hbm, out_hbm,
                  src0_v, src1_v, dst0_v, dst1_v, val0_v, val1_v,
                  tab_sh, acc_sh, sc0, sc1):
    cid = lax.axis_index("c")
    sid = lax.axis_index("s")
    w = sid * 2 + cid
    off = sid * NPT
    pltpu.sync_copy(w_hbm.at[pl.ds(off, NPT)], tab_sh.at[pl.ds(off, NPT)])
    pltpu.sync_copy(zeros_hbm.at[pl.ds(off, NPT)], acc_sh.at[pl.ds(off, NPT)])
    plsc.subcore_barrier()

    e0 = w * EPW
    bufs = ((src0_v, dst0_v, val0_v, sc0), (src1_v, dst1_v, val1_v, sc1))

    def body(i, carry):
        for b, (src_v, dst_v, val_v, sc) in enumerate(bufs):
            @pl.when(i > 0)
            def _():
                pltpu.make_async_copy(val_v, acc_sh.at[dst_v], sc).wait()

            k0 = e0 + (2 * i + b) * C
            pltpu.sync_copy(src_hbm.at[pl.ds(k0, C)], src_v)
            pltpu.sync_copy(dst_hbm.at[pl.ds(k0, C)], dst_v)
            pltpu.sync_copy(tab_sh.at[src_v], val_v)
            pltpu.async_copy(val_v, acc_sh.at[dst_v], sc, add=True)
        return carry

    lax.fori_loop(0, NCH // 2, body, 0)
    for src_v, dst_v, val_v, sc in bufs:
        pltpu.make_async_copy(val_v, acc_sh.at[dst_v], sc).wait()

    plsc.subcore_barrier()
    pltpu.sync_copy(acc_sh.at[pl.ds(off, NPT)],
                    out_hbm.at[pl.ds(cid * NPAD + off, NPT)])


@functools.partial(
    pl.kernel,
    out_type=jax.ShapeDtypeStruct((4 * NPAD,), _f32),
    mesh=_MESH,
    scratch_types=[
        pltpu.VMEM((C,), jnp.int32),
        pltpu.VMEM((C,), jnp.int32),
        pltpu.VMEM((C,), jnp.int32),
        pltpu.VMEM((C,), jnp.int32),
        pltpu.VMEM((C,), _f32),
        pltpu.VMEM((C,), _f32),
        pltpu.VMEM((C,), _f32),
        pltpu.VMEM((C,), _f32),
        pltpu.VMEM_SHARED((NPAD,), _f32),
        pltpu.VMEM_SHARED((NPAD,), _f32),
        pltpu.VMEM_SHARED((NPAD,), _f32),
        pltpu.VMEM_SHARED((NPAD,), _f32),
        pltpu.SemaphoreType.DMA,
        pltpu.SemaphoreType.DMA,
        pltpu.SemaphoreType.DMA,
        pltpu.SemaphoreType.DMA,
        pltpu.SemaphoreType.DMA,
        pltpu.SemaphoreType.DMA,
    ],
)
def _prop2_kernel(src_hbm, dst_hbm, wa_hbm, wb_hbm, zeros_hbm, out_hbm,
                  src0_v, src1_v, dst0_v, dst1_v,
                  va0_v, va1_v, vb0_v, vb1_v,
                  taba_sh, tabb_sh, acca_sh, accb_sh,
                  sa0, sa1, sb0, sb1, sga, sgb):
    cid = lax.axis_index("c")
    sid = lax.axis_index("s")
    w = sid * 2 + cid
    off = sid * NPT
    pltpu.sync_copy(wa_hbm.at[pl.ds(off, NPT)], taba_sh.at[pl.ds(off, NPT)])
    pltpu.sync_copy(wb_hbm.at[pl.ds(off, NPT)], tabb_sh.at[pl.ds(off, NPT)])
    pltpu.sync_copy(zeros_hbm.at[pl.ds(off, NPT)], acca_sh.at[pl.ds(off, NPT)])
    pltpu.sync_copy(zeros_hbm.at[pl.ds(off, NPT)], accb_sh.at[pl.ds(off, NPT)])
    plsc.subcore_barrier()

    e0 = w * EPW
    bufs = ((src0_v, dst0_v, va0_v, vb0_v, sa0, sb0),
            (src1_v, dst1_v, va1_v, vb1_v, sa1, sb1))

    def body(i, carry):
        for b, (src_v, dst_v, va_v, vb_v, sa, sb) in enumerate(bufs):
            @pl.when(i > 0)
            def _():
                pltpu.make_async_copy(va_v, acca_sh.at[dst_v], sa).wait()
                pltpu.make_async_copy(vb_v, accb_sh.at[dst_v], sb).wait()

            k0 = e0 + (2 * i + b) * C
            pltpu.sync_copy(src_hbm.at[pl.ds(k0, C)], src_v)
            pltpu.sync_copy(dst_hbm.at[pl.ds(k0, C)], dst_v)
            ga = pltpu.async_copy(taba_sh.at[src_v], va_v, sga)
            gb = pltpu.async_copy(tabb_sh.at[src_v], vb_v, sgb)
            ga.wait()
            gb.wait()
            pltpu.async_copy(va_v, acca_sh.at[dst_v], sa, add=True)
            pltpu.async_copy(vb_v, accb_sh.at[dst_v], sb, add=True)
        return carry

    lax.fori_loop(0, NCH // 2, body, 0)
    for src_v, dst_v, va_v, vb_v, sa, sb in bufs:
        pltpu.make_async_copy(va_v, acca_sh.at[dst_v], sa).wait()
        pltpu.make_async_copy(vb_v, accb_sh.at[dst_v], sb).wait()

    plsc.subcore_barrier()
    pltpu.sync_copy(acca_sh.at[pl.ds(off, NPT)],
                    out_hbm.at[pl.ds(cid * NPAD + off, NPT)])
    pltpu.sync_copy(accb_sh.at[pl.ds(off, NPT)],
                    out_hbm.at[pl.ds((2 + cid) * NPAD + off, NPT)])


def _tc_prep_body(deg_ref, x_ref, dinv_ref, w_ref):
    d = deg_ref[0] + deg_ref[1] + 1.0
    dinv = lax.rsqrt(d)
    dinv_ref[...] = dinv
    w_ref[...] = dinv * x_ref[...]


def _tc_layer1_body(g1_ref, w_ref, dinv_ref, w1_ref, b1_ref, wa_ref, wb_ref):
    dinv = dinv_ref[...]
    p1 = dinv * (g1_ref[0] + g1_ref[1] + w_ref[...])
    ha = jnp.maximum(p1 * w1_ref[0, 0] + b1_ref[0, 0], 0.0)
    hb = jnp.maximum(p1 * w1_ref[0, 1] + b1_ref[0, 1], 0.0)
    wa_ref[...] = dinv * ha
    wb_ref[...] = dinv * hb


def _tc_final_body(g2_ref, wa_ref, wb_ref, dinv_ref, w2_ref, b2_ref,
                   oa_ref, ob_ref):
    dinv = dinv_ref[...]
    ua = dinv * (g2_ref[0] + g2_ref[1] + wa_ref[...])
    ub = dinv * (g2_ref[2] + g2_ref[3] + wb_ref[...])
    oa_ref[...] = ua * w2_ref[0, 0] + ub * w2_ref[1, 0] + b2_ref[0, 0]
    ob_ref[...] = ua * w2_ref[0, 1] + ub * w2_ref[1, 1] + b2_ref[0, 1]


def _vspec():
    return pl.BlockSpec(memory_space=pltpu.VMEM)


def _sspec():
    return pl.BlockSpec(memory_space=pltpu.SMEM)


def kernel(x, edge_index, W1, b1, W2, b2):
    n = x.shape[0]
    assert n == N_NODES and edge_index.shape[1] == N_EDGES
    src = edge_index[0].astype(jnp.int32)
    dst = edge_index[1].astype(jnp.int32)
    xp = jnp.pad(x[:, 0], (0, NPAD - n))
    zeros = jnp.zeros((NPAD,), _f32)
    ones = jnp.ones((C,), _f32)

    degp = _deg_kernel(dst, ones, zeros)

    dinv, w1v = pl.pallas_call(
        _tc_prep_body,
        out_shape=[jax.ShapeDtypeStruct((TC_R, LANES), _f32)] * 2,
        in_specs=[_vspec(), _vspec()],
        out_specs=[_vspec(), _vspec()],
    )(degp.reshape(2, TC_R, LANES), xp.reshape(TC_R, LANES))

    g1p = _prop1_kernel(src, dst, w1v.reshape(NPAD), zeros)

    w2a, w2b = pl.pallas_call(
        _tc_layer1_body,
        out_shape=[jax.ShapeDtypeStruct((TC_R, LANES), _f32)] * 2,
        in_specs=[_vspec(), _vspec(), _vspec(), _sspec(), _sspec()],
        out_specs=[_vspec(), _vspec()],
    )(g1p.reshape(2, TC_R, LANES), w1v, dinv,
      W1.reshape(1, 2), b1.reshape(1, 2))

    g2p = _prop2_kernel(src, dst, w2a.reshape(NPAD), w2b.reshape(NPAD), zeros)

    oa, ob = pl.pallas_call(
        _tc_final_body,
        out_shape=[jax.ShapeDtypeStruct((TC_R, LANES), _f32)] * 2,
        in_specs=[_vspec()] * 4 + [_sspec(), _sspec()],
        out_specs=[_vspec(), _vspec()],
    )(g2p.reshape(4, TC_R, LANES), w2a, w2b, dinv,
      W2.reshape(2, 2), b2.reshape(1, 2))

    return jnp.stack([oa.reshape(NPAD)[:n], ob.reshape(NPAD)[:n]], axis=-1)


# speedup vs baseline: 1.0165x; 1.0165x over previous
import functools

import jax
import jax.numpy as jnp
from jax import lax
from jax.experimental import pallas as pl
from jax.experimental.pallas import tpu as pltpu
from jax.experimental.pallas import tpu_sc as plsc

N_NODES = 100000
N_EDGES = 3200000

NW = 32
EPW = N_EDGES // NW
C = 10000
NCH = EPW // C

NPAD = 100352
NPT = NPAD // 16
TC_R = NPAD // 128
LANES = 128

_MESH = plsc.VectorSubcoreMesh(core_axis_name="c", subcore_axis_name="s")
_f32 = jnp.float32


@functools.partial(
    pl.kernel,
    out_type=jax.ShapeDtypeStruct((2 * NPAD,), _f32),
    mesh=_MESH,
    scratch_types=[
        pltpu.VMEM((C,), jnp.int32),
        pltpu.VMEM((C,), jnp.int32),
        pltpu.VMEM((C,), _f32),
        pltpu.VMEM_SHARED((NPAD,), _f32),
        pltpu.SemaphoreType.DMA,
        pltpu.SemaphoreType.DMA,
    ],
)
def _deg_kernel(dst_hbm, ones_hbm, zeros_hbm, out_hbm,
                idx0_v, idx1_v, ones_v, acc_sh, sc0, sc1):
    cid = lax.axis_index("c")
    sid = lax.axis_index("s")
    w = sid * 2 + cid
    off = sid * NPT
    pltpu.sync_copy(ones_hbm, ones_v)
    pltpu.sync_copy(zeros_hbm.at[pl.ds(off, NPT)], acc_sh.at[pl.ds(off, NPT)])
    plsc.subcore_barrier()

    e0 = w * EPW
    bufs = ((idx0_v, sc0), (idx1_v, sc1))

    def body(i, carry):
        for b, (idx_v, sc) in enumerate(bufs):
            @pl.when(i > 0)
            def _():
                pltpu.make_async_copy(ones_v, acc_sh.at[idx_v], sc).wait()

            pltpu.sync_copy(dst_hbm.at[pl.ds(e0 + (2 * i + b) * C, C)], idx_v)
            pltpu.async_copy(ones_v, acc_sh.at[idx_v], sc, add=True)
        return carry

    lax.fori_loop(0, NCH // 2, body, 0)
    for idx_v, sc in bufs:
        pltpu.make_async_copy(ones_v, acc_sh.at[idx_v], sc).wait()

    plsc.subcore_barrier()
    pltpu.sync_copy(acc_sh.at[pl.ds(off, NPT)],
                    out_hbm.at[pl.ds(cid * NPAD + off, NPT)])


@functools.partial(
    pl.kernel,
    out_type=jax.ShapeDtypeStruct((2 * NPAD,), _f32),
    mesh=_MESH,
    scratch_types=[
        pltpu.VMEM((C,), jnp.int32),
        pltpu.VMEM((C,), jnp.int32),
        pltpu.VMEM((C,), jnp.int32),
        pltpu.VMEM((C,), jnp.int32),
        pltpu.VMEM((C,), _f32),
        pltpu.VMEM((C,), _f32),
        pltpu.VMEM_SHARED((NPAD,), _f32),
        pltpu.VMEM_SHARED((NPAD,), _f32),
        pltpu.SemaphoreType.DMA,
        pltpu.SemaphoreType.DMA,
    ],
)
def _prop1_kernel(src_hbm, dst_hbm, w_hbm, zeros_hbm, out_hbm,
                  src0_v, src1_v, dst0_v, dst1_v, val0_v, val1_v,
                  tab_sh, acc_sh, sc0, sc1):
    cid = lax.axis_index("c")
    sid = lax.axis_index("s")
    w = sid * 2 + cid
    off = sid * NPT
    pltpu.sync_copy(w_hbm.at[pl.ds(off, NPT)], tab_sh.at[pl.ds(off, NPT)])
    pltpu.sync_copy(zeros_hbm.at[pl.ds(off, NPT)], acc_sh.at[pl.ds(off, NPT)])
    plsc.subcore_barrier()

    e0 = w * EPW
    bufs = ((src0_v, dst0_v, val0_v, sc0), (src1_v, dst1_v, val1_v, sc1))

    def body(i, carry):
        for b, (src_v, dst_v, val_v, sc) in enumerate(bufs):
            @pl.when(i > 0)
            def _():
                pltpu.make_async_copy(val_v, acc_sh.at[dst_v], sc).wait()

            k0 = e0 + (2 * i + b) * C
            pltpu.sync_copy(src_hbm.at[pl.ds(k0, C)], src_v)
            pltpu.sync_copy(dst_hbm.at[pl.ds(k0, C)], dst_v)
            pltpu.sync_copy(tab_sh.at[src_v], val_v)
            pltpu.async_copy(val_v, acc_sh.at[dst_v], sc, add=True)
        return carry

    lax.fori_loop(0, NCH // 2, body, 0)
    for src_v, dst_v, val_v, sc in bufs:
        pltpu.make_async_copy(val_v, acc_sh.at[dst_v], sc).wait()

    plsc.subcore_barrier()
    pltpu.sync_copy(acc_sh.at[pl.ds(off, NPT)],
                    out_hbm.at[pl.ds(cid * NPAD + off, NPT)])


@functools.partial(
    pl.kernel,
    out_type=jax.ShapeDtypeStruct((4 * NPAD,), _f32),
    mesh=_MESH,
    scratch_types=[
        pltpu.VMEM((C,), jnp.int32),
        pltpu.VMEM((C,), jnp.int32),
        pltpu.VMEM((C,), jnp.int32),
        pltpu.VMEM((C,), jnp.int32),
        pltpu.VMEM((C,), _f32),
        pltpu.VMEM((C,), _f32),
        pltpu.VMEM((C,), _f32),
        pltpu.VMEM((C,), _f32),
        pltpu.VMEM_SHARED((NPAD,), _f32),
        pltpu.VMEM_SHARED((NPAD,), _f32),
        pltpu.VMEM_SHARED((NPAD,), _f32),
        pltpu.VMEM_SHARED((NPAD,), _f32),
        pltpu.SemaphoreType.DMA,
        pltpu.SemaphoreType.DMA,
        pltpu.SemaphoreType.DMA,
        pltpu.SemaphoreType.DMA,
    ],
)
def _prop2_kernel(src_hbm, dst_hbm, wa_hbm, wb_hbm, zeros_hbm, out_hbm,
                  src0_v, src1_v, dst0_v, dst1_v,
                  va0_v, va1_v, vb0_v, vb1_v,
                  taba_sh, tabb_sh, acca_sh, accb_sh,
                  sa0, sa1, sb0, sb1):
    cid = lax.axis_index("c")
    sid = lax.axis_index("s")
    w = sid * 2 + cid
    off = sid * NPT
    pltpu.sync_copy(wa_hbm.at[pl.ds(off, NPT)], taba_sh.at[pl.ds(off, NPT)])
    pltpu.sync_copy(wb_hbm.at[pl.ds(off, NPT)], tabb_sh.at[pl.ds(off, NPT)])
    pltpu.sync_copy(zeros_hbm.at[pl.ds(off, NPT)], acca_sh.at[pl.ds(off, NPT)])
    pltpu.sync_copy(zeros_hbm.at[pl.ds(off, NPT)], accb_sh.at[pl.ds(off, NPT)])
    plsc.subcore_barrier()

    e0 = w * EPW
    bufs = ((src0_v, dst0_v, va0_v, vb0_v, sa0, sb0),
            (src1_v, dst1_v, va1_v, vb1_v, sa1, sb1))

    def body(i, carry):
        for b, (src_v, dst_v, va_v, vb_v, sa, sb) in enumerate(bufs):
            @pl.when(i > 0)
            def _():
                pltpu.make_async_copy(va_v, acca_sh.at[dst_v], sa).wait()
                pltpu.make_async_copy(vb_v, accb_sh.at[dst_v], sb).wait()

            k0 = e0 + (2 * i + b) * C
            pltpu.sync_copy(src_hbm.at[pl.ds(k0, C)], src_v)
            pltpu.sync_copy(dst_hbm.at[pl.ds(k0, C)], dst_v)
            pltpu.sync_copy(taba_sh.at[src_v], va_v)
            pltpu.sync_copy(tabb_sh.at[src_v], vb_v)
            pltpu.async_copy(va_v, acca_sh.at[dst_v], sa, add=True)
            pltpu.async_copy(vb_v, accb_sh.at[dst_v], sb, add=True)
        return carry

    lax.fori_loop(0, NCH // 2, body, 0)
    for src_v, dst_v, va_v, vb_v, sa, sb in bufs:
        pltpu.make_async_copy(va_v, acca_sh.at[dst_v], sa).wait()
        pltpu.make_async_copy(vb_v, accb_sh.at[dst_v], sb).wait()

    plsc.subcore_barrier()
    pltpu.sync_copy(acca_sh.at[pl.ds(off, NPT)],
                    out_hbm.at[pl.ds(cid * NPAD + off, NPT)])
    pltpu.sync_copy(accb_sh.at[pl.ds(off, NPT)],
                    out_hbm.at[pl.ds((2 + cid) * NPAD + off, NPT)])


def _tc_prep_body(deg_ref, x_ref, dinv_ref, w_ref):
    d = deg_ref[0] + deg_ref[1] + 1.0
    dinv = lax.rsqrt(d)
    dinv_ref[...] = dinv
    w_ref[...] = dinv * x_ref[...]


def _tc_layer1_body(g1_ref, w_ref, dinv_ref, w1_ref, b1_ref, wa_ref, wb_ref):
    dinv = dinv_ref[...]
    p1 = dinv * (g1_ref[0] + g1_ref[1] + w_ref[...])
    ha = jnp.maximum(p1 * w1_ref[0, 0] + b1_ref[0, 0], 0.0)
    hb = jnp.maximum(p1 * w1_ref[0, 1] + b1_ref[0, 1], 0.0)
    wa_ref[...] = dinv * ha
    wb_ref[...] = dinv * hb


def _tc_final_body(g2_ref, wa_ref, wb_ref, dinv_ref, w2_ref, b2_ref,
                   oa_ref, ob_ref):
    dinv = dinv_ref[...]
    ua = dinv * (g2_ref[0] + g2_ref[1] + wa_ref[...])
    ub = dinv * (g2_ref[2] + g2_ref[3] + wb_ref[...])
    oa_ref[...] = ua * w2_ref[0, 0] + ub * w2_ref[1, 0] + b2_ref[0, 0]
    ob_ref[...] = ua * w2_ref[0, 1] + ub * w2_ref[1, 1] + b2_ref[0, 1]


def _vspec():
    return pl.BlockSpec(memory_space=pltpu.VMEM)


def _sspec():
    return pl.BlockSpec(memory_space=pltpu.SMEM)


def kernel(x, edge_index, W1, b1, W2, b2):
    n = x.shape[0]
    assert n == N_NODES and edge_index.shape[1] == N_EDGES
    src = edge_index[0].astype(jnp.int32)
    dst = edge_index[1].astype(jnp.int32)
    xp = jnp.pad(x[:, 0], (0, NPAD - n))
    zeros = jnp.zeros((NPAD,), _f32)
    ones = jnp.ones((C,), _f32)

    degp = _deg_kernel(dst, ones, zeros)

    dinv, w1v = pl.pallas_call(
        _tc_prep_body,
        out_shape=[jax.ShapeDtypeStruct((TC_R, LANES), _f32)] * 2,
        in_specs=[_vspec(), _vspec()],
        out_specs=[_vspec(), _vspec()],
    )(degp.reshape(2, TC_R, LANES), xp.reshape(TC_R, LANES))

    g1p = _prop1_kernel(src, dst, w1v.reshape(NPAD), zeros)

    w2a, w2b = pl.pallas_call(
        _tc_layer1_body,
        out_shape=[jax.ShapeDtypeStruct((TC_R, LANES), _f32)] * 2,
        in_specs=[_vspec(), _vspec(), _vspec(), _sspec(), _sspec()],
        out_specs=[_vspec(), _vspec()],
    )(g1p.reshape(2, TC_R, LANES), w1v, dinv,
      W1.reshape(1, 2), b1.reshape(1, 2))

    g2p = _prop2_kernel(src, dst, w2a.reshape(NPAD), w2b.reshape(NPAD), zeros)

    oa, ob = pl.pallas_call(
        _tc_final_body,
        out_shape=[jax.ShapeDtypeStruct((TC_R, LANES), _f32)] * 2,
        in_specs=[_vspec()] * 4 + [_sspec(), _sspec()],
        out_specs=[_vspec(), _vspec()],
    )(g2p.reshape(4, TC_R, LANES), w2a, w2b, dinv,
      W2.reshape(2, 2), b2.reshape(1, 2))

    return jnp.stack([oa.reshape(NPAD)[:n], ob.reshape(NPAD)[:n]], axis=-1)
